# baseline (speedup 1.0000x reference)
import functools

import jax
import jax.numpy as jnp
from jax import lax
from jax.experimental import pallas as pl
from jax.experimental.pallas import tpu as pltpu
from jax.experimental.pallas import tpu_sc as plsc

N_NODES = 10000
N_EDGES = 320000
D = 128
G = 64
NC, NS, L = 2, 16, 16
NW = NC * NS
CHUNK = 128
NCH = 80
NPAIR = NCH // 2
EPW = NCH * CHUNK
EPAD = NW * EPW
NPAD = 10240
RPW = NPAD // NW
GP = 72
NPT = NPAD // NS


def _mesh():
    return plsc.VectorSubcoreMesh(
        core_axis_name="c", subcore_axis_name="s", num_cores=NC, num_subcores=NS)


_BCAST_DN = lax.GatherDimensionNumbers(
    offset_dims=(), collapsed_slice_dims=(0,), start_index_map=(0,))


def _bcast(vec, lane):
    idx = jnp.full((L, 1), lane, jnp.int32)
    return lax.gather(vec, idx, _BCAST_DN, (1,),
                      mode=lax.GatherScatterMode.PROMISE_IN_BOUNDS)


def _edge_body(x_hbm, src_hbm, dst_hbm, cci_hbm, out_hbm,
               acc_sh, cci_st, src_st, dst_st, rows_v,
               g0, g1, s0, s1, a0, a1, a2, a3, d0, d1, d2, d3, c0, c1, c2, c3):
    cid = lax.axis_index("c")
    sid = lax.axis_index("s")
    wid = sid * NC + cid
    gsem = [g0, g1]
    ssem = [s0, s1]
    asem = [a0, a1, a2, a3]
    dsem = [d0, d1, d2, d3]
    csem = [c0, c1, c2, c3]

    zv = jnp.zeros((L,), jnp.float32)

    def zrow(r, _):
        for j in range(D // L):
            rows_v[0, r, pl.ds(j * L, L)] = zv
        return 0
    lax.fori_loop(0, CHUNK, zrow, 0)
    for k in range(NPT // CHUNK):
        pltpu.async_copy(rows_v.at[0],
                         acc_sh.at[pl.ds(sid * NPT + k * CHUNK, CHUNK)],
                         gsem[k % 2])
    for k in range(NPT // CHUNK):
        pltpu.make_async_copy(rows_v.at[0],
                              acc_sh.at[pl.ds(sid * NPT + k * CHUNK, CHUNK)],
                              gsem[k % 2]).wait()
    plsc.subcore_barrier()

    def stage_pair(q, slot):
        pltpu.async_copy(src_hbm.at[wid, q], src_st.at[slot], asem[slot])
        pltpu.async_copy(dst_hbm.at[wid, q], dst_st.at[slot], dsem[slot])
        pltpu.async_copy(cci_hbm.at[wid, q], cci_st.at[slot], csem[slot])

    def wait_pair(slot):
        pltpu.make_async_copy(src_hbm.at[wid, 0], src_st.at[slot],
                              asem[slot]).wait()
        pltpu.make_async_copy(dst_hbm.at[wid, 0], dst_st.at[slot],
                              dsem[slot]).wait()
        pltpu.make_async_copy(cci_hbm.at[wid, 0], cci_st.at[slot],
                              csem[slot]).wait()

    def issue_gather(pp, k, b):
        pltpu.async_copy(x_hbm.at[src_st.at[pp, k, 0]], rows_v.at[b],
                         gsem[b])

    def wait_g(b):
        pltpu.make_async_copy(x_hbm.at[src_st.at[0, 0, 0]], rows_v.at[b],
                              gsem[b]).wait()

    def issue_scatter(pp, k, b):
        pltpu.async_copy(rows_v.at[b], acc_sh.at[dst_st.at[pp, k, 0]],
                         ssem[b], add=True)

    def wait_s(b):
        pltpu.make_async_copy(rows_v.at[b], acc_sh.at[dst_st.at[0, 0, 0]],
                              ssem[b]).wait()

    def scale(pp, k, b):
        def grp(g, _):
            for i in range(4):
                r = g * 4 + i
                cvec = cci_st[pp, k, 0, pl.ds((r // L) * L, L)]
                c = _bcast(cvec, r % L)
                for j in range(D // L):
                    sl = pl.ds(j * L, L)
                    rows_v[b, r, sl] = rows_v[b, r, sl] * c
            return 0
        lax.fori_loop(0, CHUNK // 4, grp, 0)

    def quad(q0, P, first=False, tail=False):
        A = P
        B = (P + 1) % 4
        nA = (P + 2) % 4
        nB = (P + 3) % 4
        wait_g(0)
        scale(A, 0, 0)
        issue_scatter(A, 0, 0)
        if not first:
            wait_s(1)
        if not tail:
            stage_pair(q0 + 2, nA)
            stage_pair(q0 + 3, nB)
        issue_gather(A, 1, 1)
        wait_g(1)
        scale(A, 1, 1)
        issue_scatter(A, 1, 1)
        wait_s(0)
        wait_pair(B)
        issue_gather(B, 0, 0)
        wait_g(0)
        scale(B, 0, 0)
        issue_scatter(B, 0, 0)
        wait_s(1)
        issue_gather(B, 1, 1)
        wait_g(1)
        scale(B, 1, 1)
        issue_scatter(B, 1, 1)
        if not tail:
            wait_s(0)
            wait_pair(nA)
            issue_gather(nA, 0, 0)

    stage_pair(0, 0)
    stage_pair(1, 1)
    wait_pair(0)
    issue_gather(0, 0, 0)

    quad(0, 0, first=True)
    quad(2, 2)

    NSUP = NCH // 8

    def u_body(u, _):
        quad(4 * u, 0)
        quad(4 * u + 2, 2)
        return 0
    lax.fori_loop(1, NSUP - 1, u_body, 0)

    quad(4 * (NSUP - 1), 0)
    quad(4 * (NSUP - 1) + 2, 2, tail=True)

    wait_s(0)
    wait_s(1)

    plsc.subcore_barrier()
    pltpu.sync_copy(acc_sh.at[pl.ds(sid * NPT, NPT)],
                    out_hbm.at[cid, pl.ds(sid * NPT, NPT)])


def _edge_pass(x, srcr, dstr, ccir):
    kfn = pl.kernel(
        _edge_body,
        out_type=jax.ShapeDtypeStruct((NC, NPAD, D), jnp.float32),
        mesh=_mesh(),
        scratch_types=[
            pltpu.VMEM_SHARED((NPAD, D), jnp.float32),
            pltpu.VMEM((4, 2, 1, CHUNK), jnp.float32),
            pltpu.VMEM((4, 2, 1, CHUNK), jnp.int32),
            pltpu.VMEM((4, 2, 1, CHUNK), jnp.int32),
            pltpu.VMEM((2, CHUNK, D), jnp.float32),
        ] + [pltpu.SemaphoreType.DMA] * 16,
    )
    return kfn(x, srcr, dstr, ccir)


def _layer_res_body(p_ref, w_ref, xp_ref, o_ref):
    acc = p_ref[0] + p_ref[1]
    h = jnp.dot(acc, w_ref[...], preferred_element_type=jnp.float32)
    o_ref[...] = jnp.maximum(h + xp_ref[...], 0.0)


def _layer_body(p_ref, w_ref, o_ref):
    acc = p_ref[0] + p_ref[1]
    h = jnp.dot(acc, w_ref[...], preferred_element_type=jnp.float32)
    o_ref[...] = jnp.maximum(h, 0.0)


def _layer(p, W, xprev):
    nb = 16
    rb = NPAD // nb
    in_specs = [
        pl.BlockSpec((NC, rb, D), lambda i: (0, i, 0)),
        pl.BlockSpec((D, D), lambda i: (0, 0)),
    ]
    args = [p, W]
    body = _layer_body
    if xprev is not None:
        in_specs.append(pl.BlockSpec((rb, D), lambda i: (i, 0)))
        args.append(xprev)
        body = _layer_res_body
    return pl.pallas_call(
        body,
        grid=(nb,),
        in_specs=in_specs,
        out_specs=pl.BlockSpec((rb, D), lambda i: (i, 0)),
        out_shape=jax.ShapeDtypeStruct((NPAD, D), jnp.float32),
    )(*args)


def _pool_body(x_hbm, bat_hbm, stats_hbm, cnt_hbm,
               xl_v, bat_v, sum_v, sq_v, mx_v, mn_v, cnt_v, sem):
    cid = lax.axis_index("c")
    sid = lax.axis_index("s")
    wid = sid * NC + cid

    zv = jnp.zeros((L,), jnp.float32)
    ninf = jnp.full((L,), -jnp.inf, jnp.float32)
    pinf = jnp.full((L,), jnp.inf, jnp.float32)

    def init_row(r, _):
        for j in range(D // L):
            sl = pl.ds(j * L, L)
            sum_v[r, sl] = zv
            sq_v[r, sl] = zv
            mx_v[r, sl] = ninf
            mn_v[r, sl] = pinf
        cnt_v[r, pl.ds(0, L)] = zv
        return 0
    lax.fori_loop(0, GP, init_row, 0)

    pltpu.sync_copy(x_hbm.at[pl.ds(wid * RPW, RPW)], xl_v)
    pltpu.sync_copy(bat_hbm.at[wid], bat_v)

    iota = lax.iota(jnp.int32, L)

    def grp_body(rg, _):
        bvec = bat_v[pl.ds(rg * L, L)]
        for rr in range(L):
            g = bvec[rr]
            r = rg * L + rr
            cs = pl.ds(0, L)
            cnt_v[g, cs] = cnt_v[g, cs] + 1.0
            for j in range(D // L):
                sl = pl.ds(j * L, L)
                xv = xl_v[r, sl]
                sum_v[g, sl] = sum_v[g, sl] + xv
                sq_v[g, sl] = sq_v[g, sl] + xv * xv
                mx_v[g, sl] = jnp.maximum(mx_v[g, sl], xv)
                mn_v[g, sl] = jnp.minimum(mn_v[g, sl], xv)
        return 0
    lax.fori_loop(0, RPW // L, grp_body, 0)

    pltpu.sync_copy(sum_v, stats_hbm.at[wid, 0])
    pltpu.sync_copy(sq_v, stats_hbm.at[wid, 1])
    pltpu.sync_copy(mx_v, stats_hbm.at[wid, 2])
    pltpu.sync_copy(mn_v, stats_hbm.at[wid, 3])
    pltpu.sync_copy(cnt_v, cnt_hbm.at[wid])


def _pool(x2, batp):
    kfn = pl.kernel(
        _pool_body,
        out_type=(jax.ShapeDtypeStruct((NW, 4, GP, D), jnp.float32),
                  jax.ShapeDtypeStruct((NW, GP, L), jnp.float32)),
        mesh=_mesh(),
        scratch_types=[
            pltpu.VMEM((RPW, D), jnp.float32),
            pltpu.VMEM((RPW,), jnp.int32),
            pltpu.VMEM((GP, D), jnp.float32),
            pltpu.VMEM((GP, D), jnp.float32),
            pltpu.VMEM((GP, D), jnp.float32),
            pltpu.VMEM((GP, D), jnp.float32),
            pltpu.VMEM((GP, L), jnp.float32),
            pltpu.SemaphoreType.DMA,
        ],
    )
    return kfn(x2, batp)


def _head_body(stats_ref, cnt_ref, gf_ref,
               w1_ref, b1_ref, w2_ref, b2_ref, w3_ref, b3_ref, w4_ref, b4_ref,
               o_ref):
    s = stats_ref[...]
    sums = jnp.sum(s[:, 0], axis=0)[:G]
    sq = jnp.sum(s[:, 1], axis=0)[:G]
    mx = jnp.max(s[:, 2], axis=0)[:G]
    mn = jnp.min(s[:, 3], axis=0)[:G]
    cnt = jnp.sum(cnt_ref[...], axis=0)[:G, 0:1]
    cnt = jnp.maximum(cnt, 1.0)
    avg = sums / cnt
    var = jnp.maximum(sq / cnt - avg * avg, 0.0)
    std = jnp.sqrt(var + 1e-06)
    z = jnp.concatenate([avg, std, mx, mn, gf_ref[...][:, :4]], axis=1)
    z = jnp.maximum(jnp.dot(z, w1_ref[...], preferred_element_type=jnp.float32)
                    + b1_ref[...], 0.0)
    z = jnp.maximum(jnp.dot(z, w2_ref[...], preferred_element_type=jnp.float32)
                    + b2_ref[...], 0.0)
    z = jnp.maximum(jnp.dot(z, w3_ref[...], preferred_element_type=jnp.float32)
                    + b3_ref[...], 0.0)
    z = jnp.dot(z, w4_ref[...], preferred_element_type=jnp.float32) + b4_ref[...]
    half = z.shape[1] // 2
    o_ref[...] = jnp.concatenate([z[:, :half], jnp.square(z[:, half:])], axis=1)


def _head(stats, cnts, gf, fc1_w, fc1_b, fc2_w, fc2_b, fc3_w, fc3_b, fc4_w, fc4_b):
    return pl.pallas_call(
        _head_body,
        out_shape=jax.ShapeDtypeStruct((G, 2), jnp.float32),
    )(stats, cnts, gf,
      fc1_w, fc1_b.reshape(1, -1), fc2_w, fc2_b.reshape(1, -1),
      fc3_w, fc3_b.reshape(1, -1), fc4_w, fc4_b.reshape(1, -1))


def kernel(x_0, n0_to_0, cci_0_to_0, global_feature, batch_0,
           W1, W2, fc1_w, fc1_b, fc2_w, fc2_b, fc3_w, fc3_b, fc4_w, fc4_b):
    src = n0_to_0[0].astype(jnp.int32)
    dst = n0_to_0[1].astype(jnp.int32)
    cci = cci_0_to_0.astype(jnp.float32)

    npad_e = EPAD - N_EDGES
    pidx = (jnp.arange(npad_e, dtype=jnp.int32) * 13) % N_NODES
    srcr = jnp.concatenate([src, pidx]).reshape(NW, NPAIR, 2, 1, CHUNK)
    dstr = jnp.concatenate([dst, pidx]).reshape(NW, NPAIR, 2, 1, CHUNK)
    ccir = jnp.concatenate([cci, jnp.zeros((npad_e,), jnp.float32)]
                           ).reshape(NW, NPAIR, 2, 1, CHUNK)

    x0p = jnp.pad(x_0, ((0, NPAD - N_NODES), (0, 0)))
    batp = jnp.concatenate([batch_0.astype(jnp.int32),
                            jnp.full((NPAD - N_NODES,), G, jnp.int32)]
                           ).reshape(NW, RPW)

    p1 = _edge_pass(x0p, srcr, dstr, ccir)
    x1 = _layer(p1, W1, None)
    p2 = _edge_pass(x1, srcr, dstr, ccir)
    x2 = _layer(p2, W2, x1)
    stats, cnts = _pool(x2, batp)
    return _head(stats, cnts, global_feature,
                 fc1_w, fc1_b, fc2_w, fc2_b, fc3_w, fc3_b, fc4_w, fc4_b)


# speedup vs baseline: 1.6850x; 1.6850x over previous
import functools

import jax
import jax.numpy as jnp
from jax import lax
from jax.experimental import pallas as pl
from jax.experimental.pallas import tpu as pltpu
from jax.experimental.pallas import tpu_sc as plsc

N_NODES = 10000
N_EDGES = 320000
D = 128
G = 64
NC, NS, L = 2, 16, 16
NW = NC * NS
CHUNK = 64
NCH = 159
EPW = NCH * CHUNK
NBUF = 3
EPAD = NW * EPW
NPAD = 10240
RPW = NPAD // NW
GP = 72
NPT = NPAD // NS


def _mesh():
    return plsc.VectorSubcoreMesh(
        core_axis_name="c", subcore_axis_name="s", num_cores=NC, num_subcores=NS)


SDS = NBUF + 1

_BCAST_DN = lax.GatherDimensionNumbers(
    offset_dims=(), collapsed_slice_dims=(0,), start_index_map=(0,))


def _bcast(vec, lane):
    idx = jnp.full((L, 1), lane, jnp.int32)
    return lax.gather(vec, idx, _BCAST_DN, (1,),
                      mode=lax.GatherScatterMode.PROMISE_IN_BOUNDS)


def _edge_body(x_hbm, src_hbm, dst_hbm, cci_hbm, out_hbm,
               acc_sh, cci_v, src_st, dst_st, rows_v,
               g0, g1, g2, s0, s1, s2, a0, a1, a2, d0, d1, d2):
    cid = lax.axis_index("c")
    sid = lax.axis_index("s")
    wid = sid * NC + cid
    gsem = [g0, g1, g2]
    ssem = [s0, s1, s2]
    asem = [a0, a1, a2]
    dsem = [d0, d1, d2]

    zv = jnp.zeros((L,), jnp.float32)

    def zrow(r, _):
        for j in range(D // L):
            rows_v[0, r, pl.ds(j * L, L)] = zv
        return 0
    lax.fori_loop(0, CHUNK, zrow, 0)
    zs = [g0, g1, g2]
    for k in range(NPT // CHUNK):
        pltpu.async_copy(rows_v.at[0],
                         acc_sh.at[pl.ds(sid * NPT + k * CHUNK, CHUNK)],
                         zs[k % 3])
    for k in range(NPT // CHUNK):
        pltpu.make_async_copy(rows_v.at[0],
                              acc_sh.at[pl.ds(sid * NPT + k * CHUNK, CHUNK)],
                              zs[k % 3]).wait()
    plsc.subcore_barrier()

    pltpu.sync_copy(cci_hbm.at[wid], cci_v)

    def stage_src(f, slot):
        pltpu.async_copy(src_hbm.at[wid, f], src_st.at[slot], asem[slot])

    def wait_a(slot):
        pltpu.make_async_copy(src_hbm.at[wid, 0], src_st.at[slot],
                              asem[slot]).wait()

    def stage_dst(f, slot):
        pltpu.async_copy(dst_hbm.at[wid, f], dst_st.at[slot], dsem[slot])

    def wait_d(slot):
        pltpu.make_async_copy(dst_hbm.at[wid, 0], dst_st.at[slot],
                              dsem[slot]).wait()

    def issue_gather(slot):
        pltpu.async_copy(x_hbm.at[src_st.at[slot, 0]], rows_v.at[slot],
                         gsem[slot])

    def wait_g(slot):
        pltpu.make_async_copy(x_hbm.at[src_st.at[slot, 0]], rows_v.at[slot],
                              gsem[slot]).wait()

    def issue_scatter(slot):
        pltpu.async_copy(rows_v.at[slot], acc_sh.at[dst_st.at[slot, 0]],
                         ssem[slot], add=True)

    def wait_s(slot):
        pltpu.make_async_copy(rows_v.at[slot], acc_sh.at[dst_st.at[slot, 0]],
                              ssem[slot]).wait()

    def scale(t, slot):
        def grp_body(rg, _):
            cvec = cci_v[0, pl.ds(t * CHUNK + rg * L, L)]
            for rr in range(L):
                c = _bcast(cvec, rr)
                r = rg * L + rr
                for j in range(D // L):
                    sl = pl.ds(j * L, L)
                    rows_v[slot, r, sl] = rows_v[slot, r, sl] * c
            return 0
        lax.fori_loop(0, CHUNK // L, grp_body, 0)

    for f in range(NBUF):
        stage_src(f, f)
    for f in range(NBUF - 1):
        stage_dst(f, f)
    for f in range(NBUF - 1):
        wait_a(f)
        issue_gather(f)

    def step(t, b, first=False, tail=0):
        wait_g(b)
        scale(t, b)
        wait_d(b)
        issue_scatter(b)
        if tail < 2:
            bf = (b + 2) % NBUF
            if not first:
                wait_s(bf)
            wait_a(bf)
            issue_gather(bf)
        if tail < 1:
            stage_src(t + NBUF, b)
        if tail < 2:
            stage_dst(t + 2, (b + 2) % NBUF)

    for b in range(NBUF):
        step(b, b, first=(b == 0))

    NU = NCH // NBUF

    def u_body(u, _):
        for b in range(NBUF):
            step(u * NBUF + b, b)
        return 0
    lax.fori_loop(1, NU - 1, u_body, 0)

    step(NCH - 3, 0, tail=1)
    step(NCH - 2, 1, tail=2)
    step(NCH - 1, 2, tail=2)

    for b in range(NBUF):
        wait_s(b)

    plsc.subcore_barrier()
    pltpu.sync_copy(acc_sh.at[pl.ds(sid * NPT, NPT)],
                    out_hbm.at[cid, pl.ds(sid * NPT, NPT)])


def _edge_pass(x, srcr, dstr, ccir):
    kfn = pl.kernel(
        _edge_body,
        out_type=jax.ShapeDtypeStruct((NC, NPAD, D), jnp.float32),
        mesh=_mesh(),
        scratch_types=[
            pltpu.VMEM_SHARED((NPAD, D), jnp.float32),
            pltpu.VMEM((1, EPW), jnp.float32),
            pltpu.VMEM((NBUF, 1, CHUNK), jnp.int32),
            pltpu.VMEM((NBUF, 1, CHUNK), jnp.int32),
            pltpu.VMEM((NBUF, CHUNK, D), jnp.float32),
        ] + [pltpu.SemaphoreType.DMA] * 12,
    )
    return kfn(x, srcr, dstr, ccir)


def _layer_res_body(p_ref, w_ref, xp_ref, o_ref):
    acc = p_ref[0] + p_ref[1]
    h = jnp.dot(acc, w_ref[...], preferred_element_type=jnp.float32)
    o_ref[...] = jnp.maximum(h + xp_ref[...], 0.0)


def _layer_body(p_ref, w_ref, o_ref):
    acc = p_ref[0] + p_ref[1]
    h = jnp.dot(acc, w_ref[...], preferred_element_type=jnp.float32)
    o_ref[...] = jnp.maximum(h, 0.0)


def _layer(p, W, xprev):
    nb = 16
    rb = NPAD // nb
    in_specs = [
        pl.BlockSpec((NC, rb, D), lambda i: (0, i, 0)),
        pl.BlockSpec((D, D), lambda i: (0, 0)),
    ]
    args = [p, W]
    body = _layer_body
    if xprev is not None:
        in_specs.append(pl.BlockSpec((rb, D), lambda i: (i, 0)))
        args.append(xprev)
        body = _layer_res_body
    return pl.pallas_call(
        body,
        grid=(nb,),
        in_specs=in_specs,
        out_specs=pl.BlockSpec((rb, D), lambda i: (i, 0)),
        out_shape=jax.ShapeDtypeStruct((NPAD, D), jnp.float32),
    )(*args)


def _pool_body(x_hbm, bat_hbm, stats_hbm, cnt_hbm,
               xl_v, bat_v, sum_v, sq_v, mx_v, mn_v, cnt_v, sem):
    cid = lax.axis_index("c")
    sid = lax.axis_index("s")
    wid = sid * NC + cid

    zv = jnp.zeros((L,), jnp.float32)
    ninf = jnp.full((L,), -jnp.inf, jnp.float32)
    pinf = jnp.full((L,), jnp.inf, jnp.float32)

    def init_row(r, _):
        for j in range(D // L):
            sl = pl.ds(j * L, L)
            sum_v[r, sl] = zv
            sq_v[r, sl] = zv
            mx_v[r, sl] = ninf
            mn_v[r, sl] = pinf
        cnt_v[r, pl.ds(0, L)] = zv
        return 0
    lax.fori_loop(0, GP, init_row, 0)

    pltpu.sync_copy(x_hbm.at[pl.ds(wid * RPW, RPW)], xl_v)
    pltpu.sync_copy(bat_hbm.at[wid], bat_v)

    iota = lax.iota(jnp.int32, L)

    def grp_body(rg, _):
        bvec = bat_v[pl.ds(rg * L, L)]
        for rr in range(L):
            g = bvec[rr]
            r = rg * L + rr
            cs = pl.ds(0, L)
            cnt_v[g, cs] = cnt_v[g, cs] + 1.0
            for j in range(D // L):
                sl = pl.ds(j * L, L)
                xv = xl_v[r, sl]
                sum_v[g, sl] = sum_v[g, sl] + xv
                sq_v[g, sl] = sq_v[g, sl] + xv * xv
                mx_v[g, sl] = jnp.maximum(mx_v[g, sl], xv)
                mn_v[g, sl] = jnp.minimum(mn_v[g, sl], xv)
        return 0
    lax.fori_loop(0, RPW // L, grp_body, 0)

    pltpu.sync_copy(sum_v, stats_hbm.at[wid, 0])
    pltpu.sync_copy(sq_v, stats_hbm.at[wid, 1])
    pltpu.sync_copy(mx_v, stats_hbm.at[wid, 2])
    pltpu.sync_copy(mn_v, stats_hbm.at[wid, 3])
    pltpu.sync_copy(cnt_v, cnt_hbm.at[wid])


def _pool(x2, batp):
    kfn = pl.kernel(
        _pool_body,
        out_type=(jax.ShapeDtypeStruct((NW, 4, GP, D), jnp.float32),
                  jax.ShapeDtypeStruct((NW, GP, L), jnp.float32)),
        mesh=_mesh(),
        scratch_types=[
            pltpu.VMEM((RPW, D), jnp.float32),
            pltpu.VMEM((RPW,), jnp.int32),
            pltpu.VMEM((GP, D), jnp.float32),
            pltpu.VMEM((GP, D), jnp.float32),
            pltpu.VMEM((GP, D), jnp.float32),
            pltpu.VMEM((GP, D), jnp.float32),
            pltpu.VMEM((GP, L), jnp.float32),
            pltpu.SemaphoreType.DMA,
        ],
    )
    return kfn(x2, batp)


def _head_body(stats_ref, cnt_ref, gf_ref,
               w1_ref, b1_ref, w2_ref, b2_ref, w3_ref, b3_ref, w4_ref, b4_ref,
               o_ref):
    s = stats_ref[...]
    sums = jnp.sum(s[:, 0], axis=0)[:G]
    sq = jnp.sum(s[:, 1], axis=0)[:G]
    mx = jnp.max(s[:, 2], axis=0)[:G]
    mn = jnp.min(s[:, 3], axis=0)[:G]
    cnt = jnp.sum(cnt_ref[...], axis=0)[:G, 0:1]
    cnt = jnp.maximum(cnt, 1.0)
    avg = sums / cnt
    var = jnp.maximum(sq / cnt - avg * avg, 0.0)
    std = jnp.sqrt(var + 1e-06)
    z = jnp.concatenate([avg, std, mx, mn, gf_ref[...][:, :4]], axis=1)
    z = jnp.maximum(jnp.dot(z, w1_ref[...], preferred_element_type=jnp.float32)
                    + b1_ref[...], 0.0)
    z = jnp.maximum(jnp.dot(z, w2_ref[...], preferred_element_type=jnp.float32)
                    + b2_ref[...], 0.0)
    z = jnp.maximum(jnp.dot(z, w3_ref[...], preferred_element_type=jnp.float32)
                    + b3_ref[...], 0.0)
    z = jnp.dot(z, w4_ref[...], preferred_element_type=jnp.float32) + b4_ref[...]
    half = z.shape[1] // 2
    o_ref[...] = jnp.concatenate([z[:, :half], jnp.square(z[:, half:])], axis=1)


def _head(stats, cnts, gf, fc1_w, fc1_b, fc2_w, fc2_b, fc3_w, fc3_b, fc4_w, fc4_b):
    return pl.pallas_call(
        _head_body,
        out_shape=jax.ShapeDtypeStruct((G, 2), jnp.float32),
    )(stats, cnts, gf,
      fc1_w, fc1_b.reshape(1, -1), fc2_w, fc2_b.reshape(1, -1),
      fc3_w, fc3_b.reshape(1, -1), fc4_w, fc4_b.reshape(1, -1))


def kernel(x_0, n0_to_0, cci_0_to_0, global_feature, batch_0,
           W1, W2, fc1_w, fc1_b, fc2_w, fc2_b, fc3_w, fc3_b, fc4_w, fc4_b):
    src = n0_to_0[0].astype(jnp.int32)
    dst = n0_to_0[1].astype(jnp.int32)
    cci = cci_0_to_0.astype(jnp.float32)

    npad_e = EPAD - N_EDGES
    pidx = (jnp.arange(npad_e, dtype=jnp.int32) * 13) % N_NODES
    srcr = jnp.concatenate([src, pidx]).reshape(NW, NCH, 1, CHUNK)
    dstr = jnp.concatenate([dst, pidx]).reshape(NW, NCH, 1, CHUNK)
    ccir = jnp.concatenate([cci, jnp.zeros((npad_e,), jnp.float32)]
                           ).reshape(NW, 1, EPW)

    x0p = jnp.pad(x_0, ((0, NPAD - N_NODES), (0, 0)))
    batp = jnp.concatenate([batch_0.astype(jnp.int32),
                            jnp.full((NPAD - N_NODES,), G, jnp.int32)]
                           ).reshape(NW, RPW)

    p1 = _edge_pass(x0p, srcr, dstr, ccir)
    x1 = _layer(p1, W1, None)
    p2 = _edge_pass(x1, srcr, dstr, ccir)
    x2 = _layer(p2, W2, x1)
    stats, cnts = _pool(x2, batp)
    return _head(stats, cnts, global_feature,
                 fc1_w, fc1_b, fc2_w, fc2_b, fc3_w, fc3_b, fc4_w, fc4_b)
